# Initial kernel scaffold; baseline (speedup 1.0000x reference)
import functools

import jax
import jax.numpy as jnp
from jax import lax
from jax.experimental import pallas as pl
from jax.experimental.pallas import tpu as pltpu
from jax.experimental.pallas import tpu_sc as plsc

_NC = 2
_NS = 16
_NW = _NC * _NS


@functools.lru_cache(maxsize=None)
def _make_pool(B, L, V, D):
    rows_per_w = B // _NW
    mesh = plsc.VectorSubcoreMesh(
        core_axis_name="c", subcore_axis_name="s",
        num_cores=_NC, num_subcores=_NS)

    @functools.partial(
        pl.kernel,
        out_type=jax.ShapeDtypeStruct((B, D), jnp.float32),
        mesh=mesh,
        scratch_types=[
            pltpu.VMEM((rows_per_w, L), jnp.int32),
            pltpu.VMEM((L, D), jnp.float32),
            pltpu.VMEM((rows_per_w, D), jnp.float32),
            pltpu.SemaphoreType.DMA,
        ],
    )
    def pool(x_hbm, emb_hbm, out_hbm, idx_v, rows_v, acc_v, sem):
        wid = lax.axis_index("s") * _NC + lax.axis_index("c")
        base = wid * rows_per_w
        pltpu.sync_copy(x_hbm.at[pl.ds(base, rows_per_w)], idx_v)

        def row_body(r, carry):
            pltpu.async_copy(emb_hbm.at[idx_v.at[r]], rows_v, sem).wait()

            def acc_body(j, acc):
                return acc + rows_v[j]

            acc = lax.fori_loop(0, L, acc_body,
                                jnp.zeros((D,), jnp.float32))
            acc_v[r] = acc
            return carry

        lax.fori_loop(0, rows_per_w, row_body, 0)
        pltpu.sync_copy(acc_v, out_hbm.at[pl.ds(base, rows_per_w)])

    return pool


@functools.lru_cache(maxsize=None)
def _make_fc1(B, V, D, vblk=2048):
    grid = (V + vblk - 1) // vblk

    def mm(pooled_ref, w_ref, b_ref, out_ref):
        out_ref[...] = lax.dot_general(
            pooled_ref[...], w_ref[...],
            (((1,), (1,)), ((), ())),
            preferred_element_type=jnp.float32,
        ) + b_ref[...]

    return pl.pallas_call(
        mm,
        grid=(grid,),
        in_specs=[
            pl.BlockSpec((B, D), lambda i: (0, 0)),
            pl.BlockSpec((vblk, D), lambda i: (i, 0)),
            pl.BlockSpec((1, vblk), lambda i: (0, i)),
        ],
        out_specs=pl.BlockSpec((B, vblk), lambda i: (0, i)),
        out_shape=jax.ShapeDtypeStruct((B, V), jnp.float32),
    )


def kernel(x, emb, W, b):
    B, L = x.shape
    V, D = emb.shape
    emb_eff = emb.at[0].set(0.0)
    pooled = _make_pool(B, L, V, D)(x.astype(jnp.int32), emb_eff)
    return _make_fc1(B, V, D)(pooled, W, b.reshape(1, V))


# speedup vs baseline: 1.3786x; 1.3786x over previous
import functools

import jax
import jax.numpy as jnp
from jax import lax
from jax.experimental import pallas as pl
from jax.experimental.pallas import tpu as pltpu
from jax.experimental.pallas import tpu_sc as plsc

_NC = 2
_NS = 16
_NW = _NC * _NS


_CH = 100


@functools.lru_cache(maxsize=None)
def _make_pool(B, L, V, D):
    assert L % _CH == 0
    cpr = L // _CH
    rows_per_w = B // _NW
    chunks_per_w = rows_per_w * cpr
    mesh = plsc.VectorSubcoreMesh(
        core_axis_name="c", subcore_axis_name="s",
        num_cores=_NC, num_subcores=_NS)

    @functools.partial(
        pl.kernel,
        out_type=jax.ShapeDtypeStruct((B, D), jnp.float32),
        mesh=mesh,
        scratch_types=[
            pltpu.VMEM((chunks_per_w, _CH), jnp.int32),
            pltpu.VMEM((_CH, D), jnp.float32),
            pltpu.VMEM((rows_per_w, D), jnp.float32),
            pltpu.SemaphoreType.DMA,
        ],
        compiler_params=pltpu.CompilerParams(use_tc_tiling_on_sc=False),
    )
    def pool(x_hbm, emb_hbm, out_hbm, idx_v, rows_v, acc_v, sem):
        wid = lax.axis_index("s") * _NC + lax.axis_index("c")
        base = wid * chunks_per_w
        pltpu.sync_copy(x_hbm.at[pl.ds(base, chunks_per_w)], idx_v)

        def row_body(r, carry):
            def chunk_body(h, acc):
                pltpu.async_copy(
                    emb_hbm.at[idx_v.at[r * cpr + h]], rows_v, sem).wait()

                def acc_body(j, a):
                    return a + rows_v[j]

                return lax.fori_loop(0, _CH, acc_body, acc)

            acc_v[r] = lax.fori_loop(0, cpr, chunk_body,
                                     jnp.zeros((D,), jnp.float32))
            return carry

        lax.fori_loop(0, rows_per_w, row_body, 0)
        pltpu.sync_copy(acc_v, out_hbm.at[pl.ds(wid * rows_per_w, rows_per_w)])

    return pool


@functools.lru_cache(maxsize=None)
def _make_fc1(B, V, D, vblk=2048):
    grid = (V + vblk - 1) // vblk

    def mm(pooled_ref, w_ref, b_ref, out_ref):
        out_ref[...] = lax.dot_general(
            pooled_ref[...], w_ref[...],
            (((1,), (1,)), ((), ())),
            preferred_element_type=jnp.float32,
        ) + b_ref[...]

    return pl.pallas_call(
        mm,
        grid=(grid,),
        in_specs=[
            pl.BlockSpec((B, D), lambda i: (0, 0)),
            pl.BlockSpec((vblk, D), lambda i: (i, 0)),
            pl.BlockSpec((1, vblk), lambda i: (0, i)),
        ],
        out_specs=pl.BlockSpec((B, vblk), lambda i: (0, i)),
        out_shape=jax.ShapeDtypeStruct((B, V), jnp.float32),
    )


def kernel(x, emb, W, b):
    B, L = x.shape
    V, D = emb.shape
    emb_eff = emb.at[0].set(0.0)
    x_chunks = x.astype(jnp.int32).reshape(B * L // _CH, _CH)
    pooled = _make_pool(B, L, V, D)(x_chunks, emb_eff)
    return _make_fc1(B, V, D)(pooled, W, b.reshape(1, V))


# speedup vs baseline: 1.4922x; 1.0824x over previous
import functools

import jax
import jax.numpy as jnp
from jax import lax
from jax.experimental import pallas as pl
from jax.experimental.pallas import tpu as pltpu
from jax.experimental.pallas import tpu_sc as plsc

_NC = 2
_NS = 16
_NW = _NC * _NS
_CH = 100
_LANES = 16


@functools.lru_cache(maxsize=None)
def _make_pool(B, L, V, D):
    assert L % _CH == 0 and D == _LANES
    cpr = L // _CH
    rows_per_w = B // _NW
    chunks_per_w = rows_per_w * cpr
    SCH = 8
    rows_per_sup = SCH // cpr
    nsup = chunks_per_w // SCH
    sup_rows = SCH * _CH
    vecs_per_w = rows_per_w * L // _LANES
    assert (2 * L) % _LANES == 0 and (L % _LANES) == _LANES // 2
    vecs_per_pair = 2 * L // _LANES
    mesh = plsc.VectorSubcoreMesh(
        core_axis_name="c", subcore_axis_name="s",
        num_cores=_NC, num_subcores=_NS)

    @functools.partial(
        pl.kernel,
        out_type=jax.ShapeDtypeStruct((B, D), jnp.float32),
        mesh=mesh,
        scratch_types=[
            pltpu.VMEM((chunks_per_w, _CH), jnp.int32),
            pltpu.VMEM((vecs_per_w, _LANES), jnp.int32),
            pltpu.VMEM((sup_rows, D), jnp.float32),
            pltpu.VMEM((sup_rows, D), jnp.float32),
            pltpu.VMEM((rows_per_w, D), jnp.float32),
            pltpu.VMEM((rows_per_w, _LANES), jnp.float32),
            pltpu.VMEM((1, D), jnp.float32),
            pltpu.SemaphoreType.DMA,
            pltpu.SemaphoreType.DMA,
        ],
        compiler_params=pltpu.CompilerParams(use_tc_tiling_on_sc=False,
                                             needs_layout_passes=False),
    )
    def pool(xc_hbm, xv_hbm, emb_hbm, out_hbm,
             idx_v, cidx_v, buf_a, buf_b, acc_v, cnt_v, emb0_v,
             sem_a, sem_b):
        wid = lax.axis_index("s") * _NC + lax.axis_index("c")
        pltpu.sync_copy(xc_hbm.at[pl.ds(wid * chunks_per_w, chunks_per_w)],
                        idx_v)

        bufs = (buf_a, buf_b)
        sems = (sem_a, sem_b)

        def fire(s):
            buf, sem = bufs[s % 2], sems[s % 2]
            return [
                pltpu.async_copy(
                    emb_hbm.at[idx_v.at[s * SCH + k]],
                    buf.at[pl.ds(k * _CH, _CH)], sem)
                for k in range(SCH)
            ]

        descs = fire(0)

        pltpu.sync_copy(xv_hbm.at[pl.ds(wid * vecs_per_w, vecs_per_w)],
                        cidx_v)
        pltpu.sync_copy(emb_hbm.at[pl.ds(0, 1)], emb0_v)
        lo = lax.iota(jnp.int32, _LANES) < (_LANES // 2)

        def pair_body(p, carry):
            vbase = p * vecs_per_pair
            half = vecs_per_pair // 2

            def cnt_half(vstart, n, extra_mask):
                def vbody(v, c):
                    z = cidx_v[vbase + vstart + v] == 0
                    return c + plsc.all_reduce_population_count(z)
                c = lax.fori_loop(0, n, vbody,
                                  jnp.zeros((_LANES,), jnp.int32))
                zb = cidx_v[vbase + half] == 0
                return c + plsc.all_reduce_population_count(
                    jnp.logical_and(zb, extra_mask))

            c0 = cnt_half(0, half, lo)
            c1 = cnt_half(half + 1, half, jnp.logical_not(lo))
            cnt_v[2 * p] = c0.astype(jnp.float32)
            cnt_v[2 * p + 1] = c1.astype(jnp.float32)
            return carry

        lax.fori_loop(0, rows_per_w // 2, pair_body, 0)
        emb0 = emb0_v[0]

        for s in range(nsup):
            nxt = fire(s + 1) if s + 1 < nsup else []
            for d in descs:
                d.wait()
            buf = bufs[s % 2]
            for r in range(rows_per_sup):
                def acc_body(j, a, _r=r, _buf=buf):
                    return a + _buf[_r * L + j]
                row = s * rows_per_sup + r
                acc = lax.fori_loop(0, L, acc_body,
                                    jnp.zeros((D,), jnp.float32))
                acc_v[row] = acc - cnt_v[row] * emb0
            descs = nxt
        pltpu.sync_copy(acc_v, out_hbm.at[pl.ds(wid * rows_per_w,
                                                rows_per_w)])

    return pool


@functools.lru_cache(maxsize=None)
def _make_fc1(B, V, D, vblk=2048):
    grid = (V + vblk - 1) // vblk

    def mm(pooled_ref, w_ref, b_ref, out_ref):
        out_ref[...] = lax.dot_general(
            pooled_ref[...], w_ref[...],
            (((1,), (1,)), ((), ())),
            preferred_element_type=jnp.float32,
        ) + b_ref[...]

    return pl.pallas_call(
        mm,
        grid=(grid,),
        in_specs=[
            pl.BlockSpec((B, D), lambda i: (0, 0)),
            pl.BlockSpec((vblk, D), lambda i: (i, 0)),
            pl.BlockSpec((1, vblk), lambda i: (0, i)),
        ],
        out_specs=pl.BlockSpec((B, vblk), lambda i: (0, i)),
        out_shape=jax.ShapeDtypeStruct((B, V), jnp.float32),
    )


def kernel(x, emb, W, b):
    B, L = x.shape
    V, D = emb.shape
    xi = x.astype(jnp.int32)
    x_chunks = xi.reshape(B * L // _CH, _CH)
    x_vecs = xi.reshape(B * L // _LANES, _LANES)
    pooled = _make_pool(B, L, V, D)(x_chunks, x_vecs, emb)
    return _make_fc1(B, V, D)(pooled, W, b.reshape(1, V))


# speedup vs baseline: 1.4953x; 1.0020x over previous
import functools

import jax
import jax.numpy as jnp
from jax import lax
from jax.experimental import pallas as pl
from jax.experimental.pallas import tpu as pltpu
from jax.experimental.pallas import tpu_sc as plsc

_NC = 2
_NS = 16
_NW = _NC * _NS
_LANES = 16


@functools.lru_cache(maxsize=None)
def _make_depad(V, D):
    assert V % 8 == 0 and D == _LANES
    Vo = V // 8
    vb = 1024
    grid = (Vo + vb - 1) // vb

    def dp(in_ref, out_ref):
        for k in range(8):
            out_ref[:, k * D:(k + 1) * D] = in_ref[:, k, :]

        @pl.when(pl.program_id(0) == 0)
        def _():
            out_ref[0:1, 0:D] = jnp.zeros((1, D), jnp.float32)

    return pl.pallas_call(
        dp,
        grid=(grid,),
        in_specs=[pl.BlockSpec((vb, 8, D), lambda i: (i, 0, 0))],
        out_specs=pl.BlockSpec((vb, 8 * D), lambda i: (i, 0)),
        out_shape=jax.ShapeDtypeStruct((Vo, 8 * D), jnp.float32),
    )


@functools.lru_cache(maxsize=None)
def _make_pool(B, L, V, D):
    assert D == _LANES
    CH0 = 104
    CH1 = L - CH0
    assert CH0 % 8 == 0 and CH1 % 8 == 0 and CH0 <= 128 and CH1 <= 128
    rows_per_w = B // _NW
    RPS = 4
    nsup = rows_per_w // RPS
    sup_rows = RPS * L
    UNR = 8
    assert L % UNR == 0
    mesh = plsc.VectorSubcoreMesh(
        core_axis_name="c", subcore_axis_name="s",
        num_cores=_NC, num_subcores=_NS)

    @functools.partial(
        pl.kernel,
        out_type=jax.ShapeDtypeStruct((B, D), jnp.float32),
        mesh=mesh,
        scratch_types=[
            pltpu.VMEM((rows_per_w, L), jnp.int32),
            pltpu.VMEM((sup_rows, D), jnp.float32),
            pltpu.VMEM((sup_rows, D), jnp.float32),
            pltpu.VMEM((rows_per_w, D), jnp.float32),
            pltpu.SemaphoreType.DMA,
            pltpu.SemaphoreType.DMA,
        ],
        compiler_params=pltpu.CompilerParams(use_tc_tiling_on_sc=False,
                                             needs_layout_passes=False),
    )
    def pool(x_hbm, emb_hbm, out_hbm,
             idx_v, buf_a, buf_b, acc_v, sem_a, sem_b):
        wid = lax.axis_index("s") * _NC + lax.axis_index("c")
        pltpu.sync_copy(x_hbm.at[pl.ds(wid * rows_per_w, rows_per_w)],
                        idx_v)

        bufs = (buf_a, buf_b)
        sems = (sem_a, sem_b)

        def fire(s):
            buf, sem = bufs[s % 2], sems[s % 2]
            ds = []
            for rr in range(RPS):
                r = s * RPS + rr
                ds.append(pltpu.async_copy(
                    emb_hbm.at[idx_v.at[r, pl.ds(0, CH0)]],
                    buf.at[pl.ds(rr * L, CH0)], sem))
                ds.append(pltpu.async_copy(
                    emb_hbm.at[idx_v.at[r, pl.ds(CH0, CH1)]],
                    buf.at[pl.ds(rr * L + CH0, CH1)], sem))
            return ds

        descs = fire(0)
        for s in range(nsup):
            nxt = fire(s + 1) if s + 1 < nsup else []
            for d in descs:
                d.wait()
            buf = bufs[s % 2]
            for rr in range(RPS):
                base = rr * L

                def acc_body(v, accs, _base=base, _buf=buf):
                    j = _base + v * UNR
                    return tuple(
                        accs[k] + _buf[j + k] for k in range(UNR))

                accs = lax.fori_loop(
                    0, L // UNR, acc_body,
                    tuple(jnp.zeros((D,), jnp.float32)
                          for _ in range(UNR)))
                acc = accs[0]
                for k in range(1, UNR):
                    acc = acc + accs[k]
                acc_v[s * RPS + rr] = acc
            descs = nxt
        pltpu.sync_copy(acc_v, out_hbm.at[pl.ds(wid * rows_per_w,
                                                rows_per_w)])

    return pool


@functools.lru_cache(maxsize=None)
def _make_fc1(B, V, D, vblk=2048):
    grid = (V + vblk - 1) // vblk

    def mm(pooled_ref, w_ref, b_ref, out_ref):
        out_ref[...] = lax.dot_general(
            pooled_ref[...], w_ref[...],
            (((1,), (1,)), ((), ())),
            preferred_element_type=jnp.float32,
        ) + b_ref[...]

    return pl.pallas_call(
        mm,
        grid=(grid,),
        in_specs=[
            pl.BlockSpec((B, D), lambda i: (0, 0)),
            pl.BlockSpec((vblk, D), lambda i: (i, 0)),
            pl.BlockSpec((1, vblk), lambda i: (0, i)),
        ],
        out_specs=pl.BlockSpec((B, vblk), lambda i: (0, i)),
        out_shape=jax.ShapeDtypeStruct((B, V), jnp.float32),
    )


def kernel(x, emb, W, b):
    B, L = x.shape
    V, D = emb.shape
    emb_c = _make_depad(V, D)(emb.reshape(V // 8, 8, D))
    emb_flat = emb_c.reshape(V, D)
    pooled = _make_pool(B, L, V, D)(x.astype(jnp.int32), emb_flat)
    return _make_fc1(B, V, D)(pooled, W, b.reshape(1, V))


# speedup vs baseline: 4.3289x; 2.8951x over previous
import functools

import jax
import jax.numpy as jnp
from jax import lax
from jax.experimental import pallas as pl
from jax.experimental.pallas import tpu as pltpu
from jax.experimental.pallas import tpu_sc as plsc

_NC = 2
_NS = 16
_NW = _NC * _NS
_LANES = 16


@functools.lru_cache(maxsize=None)
def _make_pool(B, L, V, D):
    assert D == _LANES
    CH0 = 104
    CH1 = L - CH0
    assert CH0 % 8 == 0 and CH1 % 8 == 0 and CH0 <= 128 and CH1 <= 128
    rows_per_w = B // _NW
    RPS = 4
    nsup = rows_per_w // RPS
    sup_rows = RPS * L
    UNR = 8
    assert L % UNR == 0
    nfull = L // _LANES
    rem = L - nfull * _LANES
    mesh = plsc.VectorSubcoreMesh(
        core_axis_name="c", subcore_axis_name="s",
        num_cores=_NC, num_subcores=_NS)

    @functools.partial(
        pl.kernel,
        out_type=jax.ShapeDtypeStruct((B, D), jnp.float32),
        mesh=mesh,
        scratch_types=[
            pltpu.VMEM((rows_per_w, L), jnp.int32),
            pltpu.VMEM((sup_rows, D), jnp.float32),
            pltpu.VMEM((sup_rows, D), jnp.float32),
            pltpu.VMEM((rows_per_w, D), jnp.float32),
            pltpu.VMEM((rows_per_w, _LANES), jnp.float32),
            pltpu.VMEM((1, D), jnp.float32),
            pltpu.SemaphoreType.DMA,
            pltpu.SemaphoreType.DMA,
        ],
        compiler_params=pltpu.CompilerParams(use_tc_tiling_on_sc=False,
                                             needs_layout_passes=False),
    )
    def pool(x_hbm, emb_hbm, out_hbm,
             idx_v, buf_a, buf_b, acc_v, cnt_v, emb0_v, sem_a, sem_b):
        wid = lax.axis_index("s") * _NC + lax.axis_index("c")
        pltpu.sync_copy(x_hbm.at[pl.ds(wid * rows_per_w, rows_per_w)],
                        idx_v)

        bufs = (buf_a, buf_b)
        sems = (sem_a, sem_b)

        def fire(s):
            buf, sem = bufs[s % 2], sems[s % 2]
            ds = []
            for rr in range(RPS):
                r = s * RPS + rr
                ds.append(pltpu.async_copy(
                    emb_hbm.at[idx_v.at[r, pl.ds(0, CH0)]],
                    buf.at[pl.ds(rr * L, CH0)], sem))
                ds.append(pltpu.async_copy(
                    emb_hbm.at[idx_v.at[r, pl.ds(CH0, CH1)]],
                    buf.at[pl.ds(rr * L + CH0, CH1)], sem))
            return ds

        descs = fire(0)

        pltpu.sync_copy(emb_hbm.at[pl.ds(0, 1)], emb0_v)
        himask = lax.iota(jnp.int32, _LANES) >= (_LANES - rem)

        def cnt_body(r, carry):
            c = jnp.zeros((_LANES,), jnp.int32)
            for k in range(nfull):
                z = idx_v[r, pl.ds(k * _LANES, _LANES)] == 0
                c = c + plsc.all_reduce_population_count(z)
            zt = idx_v[r, pl.ds(L - _LANES, _LANES)] == 0
            c = c + plsc.all_reduce_population_count(
                jnp.logical_and(zt, himask))
            cnt_v[r] = c.astype(jnp.float32)
            return carry

        lax.fori_loop(0, rows_per_w, cnt_body, 0)
        emb0 = emb0_v[0]

        for s in range(nsup):
            nxt = fire(s + 1) if s + 1 < nsup else []
            for d in descs:
                d.wait()
            buf = bufs[s % 2]
            for rr in range(RPS):
                base = rr * L

                def acc_body(v, accs, _base=base, _buf=buf):
                    j = _base + v * UNR
                    return tuple(
                        accs[k] + _buf[j + k] for k in range(UNR))

                accs = lax.fori_loop(
                    0, L // UNR, acc_body,
                    tuple(jnp.zeros((D,), jnp.float32)
                          for _ in range(UNR)))
                acc = accs[0]
                for k in range(1, UNR):
                    acc = acc + accs[k]
                row = s * RPS + rr
                acc_v[row] = acc - cnt_v[row] * emb0
            descs = nxt
        pltpu.sync_copy(acc_v, out_hbm.at[pl.ds(wid * rows_per_w,
                                                rows_per_w)])

    return pool


@functools.lru_cache(maxsize=None)
def _make_fc1(B, V, D, vblk=2048):
    grid = (V + vblk - 1) // vblk

    def mm(wt_ref, pooled_ref, b_ref, out_ref):
        out_ref[...] = lax.dot_general(
            wt_ref[...], pooled_ref[...],
            (((0,), (1,)), ((), ())),
            preferred_element_type=jnp.float32,
        ) + b_ref[...].T

    return pl.pallas_call(
        mm,
        grid=(grid,),
        in_specs=[
            pl.BlockSpec((D, vblk), lambda i: (0, i)),
            pl.BlockSpec((B, D), lambda i: (0, 0)),
            pl.BlockSpec((1, vblk), lambda i: (0, i)),
        ],
        out_specs=pl.BlockSpec((vblk, B), lambda i: (i, 0)),
        out_shape=jax.ShapeDtypeStruct((V, B), jnp.float32),
    )


def kernel(x, emb, W, b):
    B, L = x.shape
    V, D = emb.shape
    pooled = _make_pool(B, L, V, D)(x.astype(jnp.int32), emb)
    logits_t = _make_fc1(B, V, D)(W.T, pooled, b.reshape(1, V))
    return logits_t.T
